# baseline (speedup 1.0000x reference)
import functools

import jax
import jax.numpy as jnp
from jax import lax
from jax.experimental import pallas as pl
from jax.experimental.pallas import tpu as pltpu
from jax.experimental.pallas import tpu_sc as plsc

BATCH = 16384
LATENT = 64
NC = 2
NS = 16
NW = NC * NS
B_PER_W = BATCH // NW
GCHUNK = 128
LANES = 16


def _make_kernel():
    mesh = plsc.VectorSubcoreMesh(
        core_axis_name="c", subcore_axis_name="s", num_cores=NC, num_subcores=NS
    )

    @functools.partial(
        pl.kernel,
        out_type=jax.ShapeDtypeStruct((BATCH,), jnp.float32),
        mesh=mesh,
        scratch_types=[
            pltpu.VMEM((B_PER_W,), jnp.int32),
            pltpu.VMEM((B_PER_W,), jnp.int32),
            pltpu.VMEM((B_PER_W, LATENT), jnp.float32),
            pltpu.VMEM((B_PER_W, LATENT), jnp.float32),
            pltpu.VMEM((B_PER_W,), jnp.float32),
            pltpu.SemaphoreType.DMA,
        ],
        compiler_params=pltpu.CompilerParams(
            needs_layout_passes=False, use_tc_tiling_on_sc=False
        ),
    )
    def svd_dot(u_hbm, m_hbm, ut_hbm, mt_hbm, out_hbm,
                uidx, midx, urows, mrows, outv, sem):
        wid = lax.axis_index("s") * NC + lax.axis_index("c")
        base = wid * B_PER_W

        pltpu.sync_copy(u_hbm.at[pl.ds(base, B_PER_W)], uidx)
        pltpu.sync_copy(m_hbm.at[pl.ds(base, B_PER_W)], midx)

        copies = []
        for j in range(B_PER_W // GCHUNK):
            sl = pl.ds(j * GCHUNK, GCHUNK)
            copies.append(pltpu.async_copy(ut_hbm.at[uidx.at[sl]], urows.at[sl], sem))
            copies.append(pltpu.async_copy(mt_hbm.at[midx.at[sl]], mrows.at[sl], sem))
        for c in copies:
            c.wait()

        def group_body(g, carry):
            rows = g * LANES + lax.iota(jnp.int32, LANES)

            def d_body(d, acc):
                col = jnp.full((LANES,), 0, jnp.int32) + d
                uv = plsc.load_gather(urows, [rows, col])
                mv = plsc.load_gather(mrows, [rows, col])
                return acc + uv * mv

            acc = lax.fori_loop(0, LATENT, d_body, jnp.zeros((LANES,), jnp.float32))
            outv[pl.ds(g * LANES, LANES)] = acc
            return carry

        lax.fori_loop(0, B_PER_W // LANES, group_body, 0)
        pltpu.sync_copy(outv, out_hbm.at[pl.ds(base, B_PER_W)])

    return svd_dot


_svd_dot = _make_kernel()


@jax.jit
def kernel(u, m, user_table, movie_table):
    out = _svd_dot(u.astype(jnp.int32), m.astype(jnp.int32),
                   user_table, movie_table)
    return out.reshape(BATCH, 1)


# speedup vs baseline: 2.1929x; 2.1929x over previous
import functools

import jax
import jax.numpy as jnp
from jax import lax
from jax.experimental import pallas as pl
from jax.experimental.pallas import tpu as pltpu
from jax.experimental.pallas import tpu_sc as plsc

BATCH = 16384
LATENT = 64
GROUP = 8
PADL = 128
NC = 2
NS = 16
NW = NC * NS
BPW = BATCH // NW
LANES = 16
NCHUNK = BPW // LANES


def _make_kernel():
    mesh = plsc.VectorSubcoreMesh(
        core_axis_name="c", subcore_axis_name="s", num_cores=NC, num_subcores=NS
    )

    slab = pltpu.VMEM((LANES, GROUP, LATENT), jnp.float32)

    @functools.partial(
        pl.kernel,
        out_type=jax.ShapeDtypeStruct((BATCH,), jnp.float32),
        mesh=mesh,
        scratch_types=[
            pltpu.VMEM((BPW,), jnp.int32),
            pltpu.VMEM((BPW,), jnp.int32),
            slab, slab,
            slab, slab,
            pltpu.VMEM((BPW,), jnp.float32),
            pltpu.SemaphoreType.DMA,
            pltpu.SemaphoreType.DMA,
            pltpu.SemaphoreType.DMA,
            pltpu.SemaphoreType.DMA,
        ],
        compiler_params=pltpu.CompilerParams(needs_layout_passes=False),
    )
    def svd_dot(u_hbm, m_hbm, ut_hbm, mt_hbm, out_hbm,
                uidx, midx, ugrp0, ugrp1, mgrp0, mgrp1, outv,
                sem_u0, sem_u1, sem_m0, sem_m1):
        wid = lax.axis_index("s") * NC + lax.axis_index("c")
        base = wid * BPW

        pltpu.sync_copy(u_hbm.at[pl.ds(base, BPW)], uidx)
        pltpu.sync_copy(m_hbm.at[pl.ds(base, BPW)], midx)

        def fire(c, ugrp, mgrp, sem_u, sem_m):
            sl = pl.ds(c * LANES, LANES)
            gu = lax.shift_right_logical(uidx[sl], 3)
            gm = lax.shift_right_logical(midx[sl], 3)
            for j in range(LANES):
                pltpu.async_copy(ut_hbm.at[gu[j]],
                                 ugrp.at[j], sem_u)
                pltpu.async_copy(mt_hbm.at[gm[j]],
                                 mgrp.at[j], sem_m)

        def drain(ugrp, mgrp, sem_u, sem_m):
            for j in range(LANES):
                pltpu.make_async_copy(
                    ut_hbm.at[0], ugrp.at[j], sem_u
                ).wait()
                pltpu.make_async_copy(
                    mt_hbm.at[0], mgrp.at[j], sem_m
                ).wait()

        def compute(c, ugrp, mgrp):
            sl = pl.ds(c * LANES, LANES)
            item = lax.iota(jnp.int32, LANES)
            urow = jnp.bitwise_and(uidx[sl], 7)
            mrow = jnp.bitwise_and(midx[sl], 7)
            acc = jnp.zeros((LANES,), jnp.float32)
            for d in range(LATENT):
                col = jnp.full((LANES,), d, jnp.int32)
                uv = plsc.load_gather(ugrp, [item, urow, col])
                mv = plsc.load_gather(mgrp, [item, mrow, col])
                acc = acc + uv * mv
            outv[sl] = acc

        fire(0, ugrp0, mgrp0, sem_u0, sem_m0)
        fire(1, ugrp1, mgrp1, sem_u1, sem_m1)

        def step(k, carry):
            c0 = 2 * k
            c1 = 2 * k + 1
            drain(ugrp0, mgrp0, sem_u0, sem_m0)
            compute(c0, ugrp0, mgrp0)

            @pl.when(k < NCHUNK // 2 - 1)
            def _():
                fire(c0 + 2, ugrp0, mgrp0, sem_u0, sem_m0)

            drain(ugrp1, mgrp1, sem_u1, sem_m1)
            compute(c1, ugrp1, mgrp1)

            @pl.when(k < NCHUNK // 2 - 1)
            def _():
                fire(c1 + 2, ugrp1, mgrp1, sem_u1, sem_m1)

            return carry

        lax.fori_loop(0, NCHUNK // 2, step, 0)
        pltpu.sync_copy(outv, out_hbm.at[pl.ds(base, BPW)])

    return svd_dot


_svd_dot = _make_kernel()


@jax.jit
def kernel(u, m, user_table, movie_table):
    users, latent = user_table.shape
    movies, _ = movie_table.shape
    ut3 = user_table.reshape(users // GROUP, GROUP, latent)
    mt3 = movie_table.reshape(movies // GROUP, GROUP, latent)
    out = _svd_dot(u.astype(jnp.int32), m.astype(jnp.int32), ut3, mt3)
    return out.reshape(BATCH, 1)
